# baseline (device time: 694161 ns/iter reference)
import jax
import jax.numpy as jnp
from jax import lax
from jax.experimental import pallas as pl
from jax.experimental.pallas import tpu as pltpu

N_DEV = 8
N_LOCAL_E = 8
CAP = 128


def _a2a_kernel(n_tok, d_model):
    def body(xs_ref, ew_ref, yret_ref, xrecv_ref, ysend_ref,
             xa_vmem, w_vmem, ya_vmem, local_sem, xa_sem, w_sem, ya_sem,
             send1, recv1, send2, recv2):
        my = lax.axis_index("i")

        disp = []
        for k in range(1, N_DEV):
            dest = lax.rem(my + k, N_DEV)
            rdma = pltpu.make_async_remote_copy(
                src_ref=xs_ref.at[dest],
                dst_ref=xrecv_ref.at[my],
                send_sem=send1.at[k - 1],
                recv_sem=recv1.at[k - 1],
                device_id=(dest,),
                device_id_type=pl.DeviceIdType.MESH,
            )
            rdma.start()
            disp.append(rdma)
        cp = pltpu.make_async_copy(xs_ref.at[my], xrecv_ref.at[my], local_sem)
        cp.start()
        cp.wait()
        for rdma in disp:
            rdma.wait_recv()

        def expert_step(s, carry):
            cw = pltpu.make_async_copy(ew_ref.at[s], w_vmem, w_sem)
            cw.start()
            cx = pltpu.make_async_copy(
                xrecv_ref.at[:, s], xa_vmem, xa_sem
            )
            cx.start()
            cx.wait()
            cw.wait()
            xa = xa_vmem[:, :, :].reshape(N_DEV * CAP, d_model)
            y = jnp.dot(xa, w_vmem[:, :], preferred_element_type=jnp.float32)
            ya_vmem[:, :, :] = y.astype(jnp.bfloat16).reshape(
                N_DEV, CAP, d_model
            )
            cy = pltpu.make_async_copy(ya_vmem, ysend_ref.at[:, s], ya_sem)
            cy.start()
            cy.wait()
            return carry

        lax.fori_loop(0, N_LOCAL_E, expert_step, 0)

        ret = []
        for k in range(1, N_DEV):
            dest = lax.rem(my + k, N_DEV)
            rdma = pltpu.make_async_remote_copy(
                src_ref=ysend_ref.at[dest],
                dst_ref=yret_ref.at[my],
                send_sem=send2.at[k - 1],
                recv_sem=recv2.at[k - 1],
                device_id=(dest,),
                device_id_type=pl.DeviceIdType.MESH,
            )
            rdma.start()
            ret.append(rdma)
        cp = pltpu.make_async_copy(ysend_ref.at[my], yret_ref.at[my], local_sem)
        cp.start()
        cp.wait()
        for rdma in disp:
            rdma.wait_send()
        for rdma in ret:
            rdma.wait_send()
            rdma.wait_recv()

    return pl.pallas_call(
        body,
        out_shape=(
            jax.ShapeDtypeStruct((N_DEV, N_LOCAL_E, CAP, d_model), jnp.bfloat16),
            jax.ShapeDtypeStruct((N_DEV, N_LOCAL_E, CAP, d_model), jnp.bfloat16),
            jax.ShapeDtypeStruct((N_DEV, N_LOCAL_E, CAP, d_model), jnp.bfloat16),
        ),
        in_specs=[
            pl.BlockSpec(memory_space=pl.ANY),
            pl.BlockSpec(memory_space=pl.ANY),
        ],
        out_specs=(
            pl.BlockSpec(memory_space=pl.ANY),
            pl.BlockSpec(memory_space=pl.ANY),
            pl.BlockSpec(memory_space=pl.ANY),
        ),
        scratch_shapes=[
            pltpu.VMEM((N_DEV, CAP, d_model), jnp.bfloat16),
            pltpu.VMEM((d_model, d_model), jnp.bfloat16),
            pltpu.VMEM((N_DEV, CAP, d_model), jnp.bfloat16),
            pltpu.SemaphoreType.DMA,
            pltpu.SemaphoreType.DMA,
            pltpu.SemaphoreType.DMA,
            pltpu.SemaphoreType.DMA,
            pltpu.SemaphoreType.DMA((N_DEV - 1,)),
            pltpu.SemaphoreType.DMA((N_DEV - 1,)),
            pltpu.SemaphoreType.DMA((N_DEV - 1,)),
            pltpu.SemaphoreType.DMA((N_DEV - 1,)),
        ],
        compiler_params=pltpu.CompilerParams(has_side_effects=True),
    )


def kernel(x, router_W, route_idx, expert_W):
    n_tok, d_model = x.shape
    n_exp = router_W.shape[1]

    scores = jnp.dot(x, router_W, preferred_element_type=jnp.float32)
    p = jax.nn.softmax(scores, axis=-1)
    g = jnp.take_along_axis(p, route_idx, axis=1)
    g = g / jnp.sum(g, axis=1, keepdims=True)

    flat_e = route_idx.reshape(-1)
    flat_t = jnp.arange(2 * n_tok, dtype=jnp.int32) // 2
    flat_g = g.reshape(-1)
    order = jnp.argsort(flat_e, stable=True)
    se = flat_e[order]
    st = flat_t[order]
    sg = flat_g[order]
    start = jnp.searchsorted(se, jnp.arange(n_exp, dtype=se.dtype))
    pos = jnp.arange(2 * n_tok, dtype=jnp.int32) - start[se]
    slot = jnp.where(pos < CAP, se * CAP + pos, n_exp * CAP)

    xs_flat = jnp.zeros((n_exp * CAP, d_model), jnp.bfloat16)
    xs_flat = xs_flat.at[slot].set(x[st].astype(jnp.bfloat16), mode="drop")
    tok_slot = jnp.full((n_exp * CAP,), n_tok, jnp.int32)
    tok_slot = tok_slot.at[slot].set(st, mode="drop")
    g_slot = jnp.zeros((n_exp * CAP,), jnp.float32)
    g_slot = g_slot.at[slot].set(sg, mode="drop")

    xsend = xs_flat.reshape(N_DEV, N_LOCAL_E, CAP, d_model)
    yret, _, _ = _a2a_kernel(n_tok, d_model)(
        xsend, expert_W.astype(jnp.bfloat16)
    )

    out = jnp.zeros((n_tok, d_model), jnp.float32)
    out = out.at[tok_slot].add(
        g_slot[:, None] * yret.reshape(n_exp * CAP, d_model).astype(jnp.float32),
        mode="drop",
    )
    return out
